# baseline (speedup 1.0000x reference)
import functools

import jax
import jax.numpy as jnp
from jax import lax
from jax.experimental import pallas as pl
from jax.experimental.pallas import tpu as pltpu


_LANES = 128


_SUBL = 8


def _distance_body(x_ref, w_ref, meant_ref, idx_ref,
                   preds2_ref, q2_ref, pm2_ref, accv_ref, acci_ref,
                   *, cb, c, nb):
    j = pl.program_id(0)
    b = x_ref.shape[0]
    ng = cb // _LANES

    @pl.when(j == 0)
    def _init():
        p = jnp.dot(x_ref[...], w_ref[...], preferred_element_type=jnp.float32)
        preds2_ref[...] = p + p
        q2_ref[...] = jnp.sum(p * p, axis=1, keepdims=True)
        accv_ref[...] = jnp.full((b, _LANES), jnp.inf, jnp.float32)
        acci_ref[...] = jnp.zeros((b, _LANES), jnp.int32)

    mt = meant_ref[...]
    pm2_ref[...] = jnp.dot(preds2_ref[...], mt,
                           preferred_element_type=jnp.float32)
    k2b = jnp.broadcast_to(jnp.sum(mt * mt, axis=0, keepdims=True),
                           (_SUBL, cb))

    def scan_block(masked):
        if masked:
            col = j * cb + lax.broadcasted_iota(jnp.int32, (1, cb), 1)
            maskb = jnp.broadcast_to(col < c, (_SUBL, cb))

        def band(i, carry):
            sl = pl.ds(i * _SUBL, _SUBL)
            pmb = pm2_ref[sl, :]
            q2b = jnp.broadcast_to(q2_ref[sl, :], (_SUBL, cb))
            d2b = (q2b + k2b) - pmb
            if masked:
                d2b = jnp.where(maskb, d2b, jnp.inf)
            m = d2b[:, 0:_LANES]
            gv = jnp.zeros((_SUBL, _LANES), jnp.int32)
            for g in range(1, ng):
                d2g = d2b[:, g * _LANES:(g + 1) * _LANES]
                better = d2g < m
                m = jnp.where(better, d2g, m)
                gv = jnp.where(better, jnp.int32(g), gv)
            accb = accv_ref[sl, :]
            better = m < accb
            accv_ref[sl, :] = jnp.where(better, m, accb)
            acci_ref[sl, :] = jnp.where(better, j * ng + gv, acci_ref[sl, :])
            return carry

        lax.fori_loop(0, b // _SUBL, band, 0, unroll=2)

    @pl.when(j < nb - 1)
    def _full_block():
        scan_block(False)

    @pl.when(j == nb - 1)
    def _last_block():
        scan_block(True)
        sv = jnp.sqrt(jnp.maximum(accv_ref[...], 0.0))
        cols = acci_ref[...] * _LANES + lax.broadcasted_iota(
            jnp.int32, (b, _LANES), 1)
        svmin = jnp.min(sv, axis=1, keepdims=True)
        cand = jnp.where(sv == svmin, cols, jnp.int32(2**30))
        idx_ref[...] = jnp.min(cand, axis=1, keepdims=True)


def _distance_argmin(x, W, mean_t, cb):
    b, d = x.shape
    c = mean_t.shape[1]
    nb = (c + cb - 1) // cb
    body = functools.partial(_distance_body, cb=cb, c=c, nb=nb)
    return pl.pallas_call(
        body,
        grid=(nb,),
        in_specs=[
            pl.BlockSpec((b, d), lambda j: (0, 0)),
            pl.BlockSpec((d, d), lambda j: (0, 0)),
            pl.BlockSpec((d, cb), lambda j: (0, j)),
        ],
        out_specs=pl.BlockSpec((b, 1), lambda j: (0, 0)),
        out_shape=jax.ShapeDtypeStruct((b, 1), jnp.int32),
        scratch_shapes=[
            pltpu.VMEM((b, d), jnp.float32),
            pltpu.VMEM((b, 1), jnp.float32),
            pltpu.VMEM((b, cb), jnp.float32),
            pltpu.VMEM((b, _LANES), jnp.float32),
            pltpu.VMEM((b, _LANES), jnp.int32),
        ],
    )(x, W, mean_t)


def _onehot_body(idx_ref, out_ref, *, cb):
    j = pl.program_id(0)
    rows = j * cb + lax.broadcasted_iota(jnp.int32, (cb, 1), 0)
    onehot = rows == idx_ref[...]
    out_ref[...] = onehot.astype(jnp.float32)


def _onehot_t(idx_row, c, cb):
    b = idx_row.shape[1]
    nb = (c + cb - 1) // cb
    body = functools.partial(_onehot_body, cb=cb)
    return pl.pallas_call(
        body,
        grid=(nb,),
        in_specs=[pl.BlockSpec((1, b), lambda j: (0, 0))],
        out_specs=pl.BlockSpec((cb, b), lambda j: (j, 0)),
        out_shape=jax.ShapeDtypeStruct((c, b), jnp.float32),
    )(idx_row)


def kernel(x, W, mean_features):
    b = x.shape[0]
    c = mean_features.shape[0]
    idx = _distance_argmin(x, W, mean_features.T, cb=1024)
    out_t = _onehot_t(idx.reshape(1, b), c, cb=2048)
    return out_t.T


# speedup vs baseline: 2.4554x; 2.4554x over previous
import functools

import jax
import jax.numpy as jnp
from jax import lax
from jax.experimental import pallas as pl
from jax.experimental.pallas import tpu as pltpu


_LANES = 128


_SUBL = 8


def _distance_body(x_ref, w_ref, meant_ref, idx_ref,
                   preds2_ref, q2_ref, accv_ref, acci_ref,
                   *, cb, c, nb):
    j = pl.program_id(0)
    b = x_ref.shape[0]
    ng = cb // _LANES

    @pl.when(j == 0)
    def _init():
        p = jnp.dot(x_ref[...], w_ref[...], preferred_element_type=jnp.float32)
        preds2_ref[...] = p + p
        q2_ref[...] = jnp.sum(p * p, axis=1, keepdims=True)
        accv_ref[...] = jnp.full((b, _LANES), jnp.inf, jnp.float32)
        acci_ref[...] = jnp.zeros((b, _LANES), jnp.int32)

    mt = meant_ref[...]
    pm2 = jnp.dot(preds2_ref[...], mt, preferred_element_type=jnp.float32)
    k2 = jnp.sum(mt * mt, axis=0, keepdims=True)
    q2b = jnp.broadcast_to(q2_ref[...], (b, _LANES))

    def scan_block(masked):
        m = None
        gv = None
        for g in range(ng):
            sl = slice(g * _LANES, (g + 1) * _LANES)
            d2g = (q2b + k2[:, sl]) - pm2[:, sl]
            if masked:
                colg = (j * cb + g * _LANES
                        + lax.broadcasted_iota(jnp.int32, (1, _LANES), 1))
                d2g = jnp.where(jnp.broadcast_to(colg < c, d2g.shape),
                                d2g, jnp.inf)
            if g == 0:
                m = d2g
                gv = jnp.zeros((b, _LANES), jnp.int32)
            else:
                better = d2g < m
                m = jnp.where(better, d2g, m)
                gv = jnp.where(better, jnp.int32(g), gv)
        better = m < accv_ref[...]
        accv_ref[...] = jnp.where(better, m, accv_ref[...])
        acci_ref[...] = jnp.where(better, j * ng + gv, acci_ref[...])

    @pl.when(j < nb - 1)
    def _full_block():
        scan_block(False)

    @pl.when(j == nb - 1)
    def _last_block():
        scan_block(True)
        sv = jnp.sqrt(jnp.maximum(accv_ref[...], 0.0))
        cols = acci_ref[...] * _LANES + lax.broadcasted_iota(
            jnp.int32, (b, _LANES), 1)
        svmin = jnp.min(sv, axis=1, keepdims=True)
        cand = jnp.where(sv == svmin, cols, jnp.int32(2**30))
        idx_ref[...] = jnp.min(cand, axis=1, keepdims=True)


def _distance_argmin(x, W, mean_t, cb):
    b, d = x.shape
    c = mean_t.shape[1]
    nb = (c + cb - 1) // cb
    body = functools.partial(_distance_body, cb=cb, c=c, nb=nb)
    return pl.pallas_call(
        body,
        grid=(nb,),
        in_specs=[
            pl.BlockSpec((b, d), lambda j: (0, 0)),
            pl.BlockSpec((d, d), lambda j: (0, 0)),
            pl.BlockSpec((d, cb), lambda j: (0, j)),
        ],
        out_specs=pl.BlockSpec((b, 1), lambda j: (0, 0)),
        out_shape=jax.ShapeDtypeStruct((b, 1), jnp.int32),
        scratch_shapes=[
            pltpu.VMEM((b, d), jnp.float32),
            pltpu.VMEM((b, 1), jnp.float32),
            pltpu.VMEM((b, _LANES), jnp.float32),
            pltpu.VMEM((b, _LANES), jnp.int32),
        ],
    )(x, W, mean_t)


def _onehot_body(idx_ref, out_ref, *, cb):
    j = pl.program_id(0)
    rows = j * cb + lax.broadcasted_iota(jnp.int32, (cb, 1), 0)
    onehot = rows == idx_ref[...]
    out_ref[...] = onehot.astype(jnp.float32)


def _onehot_t(idx_row, c, cb):
    b = idx_row.shape[1]
    nb = (c + cb - 1) // cb
    body = functools.partial(_onehot_body, cb=cb)
    return pl.pallas_call(
        body,
        grid=(nb,),
        in_specs=[pl.BlockSpec((1, b), lambda j: (0, 0))],
        out_specs=pl.BlockSpec((cb, b), lambda j: (j, 0)),
        out_shape=jax.ShapeDtypeStruct((c, b), jnp.float32),
    )(idx_row)


def kernel(x, W, mean_features):
    b = x.shape[0]
    c = mean_features.shape[0]
    idx = _distance_argmin(x, W, mean_features.T, cb=2048)
    out_t = _onehot_t(idx.reshape(1, b), c, cb=2048)
    return out_t.T


# speedup vs baseline: 2.7412x; 1.1164x over previous
import functools

import jax
import jax.numpy as jnp
from jax import lax
from jax.experimental import pallas as pl
from jax.experimental.pallas import tpu as pltpu


_LANES = 128


_SUBL = 8


def _distance_body(x_ref, w_ref, meant_ref, idx_ref,
                   preds2_ref, q2_ref, accv_ref, acci_ref,
                   *, cb, c, nb):
    j = pl.program_id(0)
    b = x_ref.shape[0]
    ng = cb // _LANES

    @pl.when(j == 0)
    def _init():
        p = jnp.dot(x_ref[...], w_ref[...], preferred_element_type=jnp.float32)
        preds2_ref[...] = p + p
        q2_ref[...] = jnp.sum(p * p, axis=1, keepdims=True)
        accv_ref[...] = jnp.full((b, _LANES), jnp.inf, jnp.float32)
        acci_ref[...] = jnp.zeros((b, _LANES), jnp.int32)

    mt = meant_ref[...]
    pm2 = jnp.dot(preds2_ref[...], mt, preferred_element_type=jnp.float32)
    k2 = jnp.sum(mt * mt, axis=0, keepdims=True)

    def scan_block(masked):
        ch = 64
        for i0 in range(0, b, ch):
            rs = slice(i0, i0 + ch)
            q2c = jnp.broadcast_to(q2_ref[rs, :], (ch, _LANES))
            m = None
            gv = None
            for g in range(ng):
                sl = slice(g * _LANES, (g + 1) * _LANES)
                d2g = (q2c + k2[:, sl]) - pm2[rs, sl]
                if masked:
                    colg = (j * cb + g * _LANES
                            + lax.broadcasted_iota(jnp.int32, (1, _LANES), 1))
                    d2g = jnp.where(jnp.broadcast_to(colg < c, d2g.shape),
                                    d2g, jnp.inf)
                if g == 0:
                    m = d2g
                    gv = jnp.zeros((ch, _LANES), jnp.int32)
                else:
                    better = d2g < m
                    m = jnp.where(better, d2g, m)
                    gv = jnp.where(better, jnp.int32(g), gv)
            accb = accv_ref[rs, :]
            better = m < accb
            accv_ref[rs, :] = jnp.where(better, m, accb)
            acci_ref[rs, :] = jnp.where(better, j * ng + gv, acci_ref[rs, :])

    @pl.when(j < nb - 1)
    def _full_block():
        scan_block(False)

    @pl.when(j == nb - 1)
    def _last_block():
        scan_block(True)
        sv = jnp.sqrt(jnp.maximum(accv_ref[...], 0.0))
        cols = acci_ref[...] * _LANES + lax.broadcasted_iota(
            jnp.int32, (b, _LANES), 1)
        svmin = jnp.min(sv, axis=1, keepdims=True)
        cand = jnp.where(sv == svmin, cols, jnp.int32(2**30))
        idx_ref[...] = jnp.min(cand, axis=1, keepdims=True)


def _distance_argmin(x, W, mean_t, cb):
    b, d = x.shape
    c = mean_t.shape[1]
    nb = (c + cb - 1) // cb
    body = functools.partial(_distance_body, cb=cb, c=c, nb=nb)
    return pl.pallas_call(
        body,
        grid=(nb,),
        in_specs=[
            pl.BlockSpec((b, d), lambda j: (0, 0)),
            pl.BlockSpec((d, d), lambda j: (0, 0)),
            pl.BlockSpec((d, cb), lambda j: (0, j)),
        ],
        out_specs=pl.BlockSpec((b, 1), lambda j: (0, 0)),
        out_shape=jax.ShapeDtypeStruct((b, 1), jnp.int32),
        scratch_shapes=[
            pltpu.VMEM((b, d), jnp.float32),
            pltpu.VMEM((b, 1), jnp.float32),
            pltpu.VMEM((b, _LANES), jnp.float32),
            pltpu.VMEM((b, _LANES), jnp.int32),
        ],
    )(x, W, mean_t)


def _onehot_body(idx_ref, out_ref, *, cb):
    j = pl.program_id(0)
    rows = j * cb + lax.broadcasted_iota(jnp.int32, (cb, 1), 0)
    onehot = rows == idx_ref[...]
    out_ref[...] = onehot.astype(jnp.float32)


def _onehot_t(idx_row, c, cb):
    b = idx_row.shape[1]
    nb = (c + cb - 1) // cb
    body = functools.partial(_onehot_body, cb=cb)
    return pl.pallas_call(
        body,
        grid=(nb,),
        in_specs=[pl.BlockSpec((1, b), lambda j: (0, 0))],
        out_specs=pl.BlockSpec((cb, b), lambda j: (j, 0)),
        out_shape=jax.ShapeDtypeStruct((c, b), jnp.float32),
    )(idx_row)


def kernel(x, W, mean_features):
    b = x.shape[0]
    c = mean_features.shape[0]
    idx = _distance_argmin(x, W, mean_features.T, cb=2048)
    out_t = _onehot_t(idx.reshape(1, b), c, cb=2048)
    return out_t.T
